# baseline (speedup 1.0000x reference)
import functools

import jax
import jax.numpy as jnp
from jax import lax
from jax.experimental import pallas as pl
from jax.experimental.pallas import tpu as pltpu
from jax.experimental.pallas import tpu_sc as plsc

N = 10000
E = 320000
D = 128
H = 64
HD = D // 2

NC = 2
NS = 16
L = 16

CHUNK = 128
NBUF = 3
NCHUNK = 158
WPE = NCHUNK * CHUNK
EPAD = WPE * NS
NP = 10240
RPT = NP // NS
DW = 8

_SC_MESH = dict(core_axis_name="c", subcore_axis_name="s")


def _make_agg():
    out_type = [jax.ShapeDtypeStruct((NC, NP, HD), jnp.float32)]
    scratch = (
        [pltpu.VMEM((NCHUNK, CHUNK), jnp.int32)] * 2
        + [pltpu.VMEM((CHUNK, HD), jnp.float32)] * NBUF
        + [pltpu.VMEM_SHARED((NP, HD), jnp.float32)]
        + [pltpu.SemaphoreType.DMA] * (2 * NBUF)
    )

    @functools.partial(
        pl.kernel, mesh=plsc.VectorSubcoreMesh(**_SC_MESH),
        out_type=out_type, scratch_types=scratch,
        compiler_params=pltpu.CompilerParams(use_tc_tiling_on_sc=False))
    def agg(table, srcs, dsts, out, src_v, dst_v, *rest):
        rows = rest[:NBUF]
        acc = rest[NBUF]
        gsem = rest[NBUF + 1:2 * NBUF + 1]
        ssem = rest[2 * NBUF + 1:]
        cid = lax.axis_index("c")
        sid = lax.axis_index("s")
        base = sid * RPT
        half = table.at[cid]

        idx_cp = (pltpu.async_copy(srcs.at[sid], src_v, gsem[0]),
                  pltpu.async_copy(dsts.at[sid], dst_v, gsem[1]))

        def zrow(r, _):
            for i in range(HD // L):
                rows[0][r, pl.ds(i * L, L)] = jnp.zeros((L,), jnp.float32)
            return 0
        lax.fori_loop(0, CHUNK, zrow, 0)
        for k in range(RPT // CHUNK):
            pltpu.sync_copy(rows[0], acc.at[pl.ds(base + k * CHUNK, CHUNK)])
        for cp in idx_cp:
            cp.wait()
        plsc.subcore_barrier()

        def g_start(j, m):
            pltpu.make_async_copy(half.at[src_v.at[j]], rows[m],
                                  gsem[m]).start()

        def g_wait(j, m):
            pltpu.make_async_copy(half.at[src_v.at[j]], rows[m],
                                  gsem[m]).wait()

        def s_add(j, m):
            pltpu.sync_copy(rows[m], acc.at[dst_v.at[j]], add=True)

        for m in range(NBUF - 1):
            g_start(jnp.int32(m), m)

        def body(jj, _):
            j0 = jj * NBUF
            for m in range(NBUF):
                j = j0 + m
                g_wait(j, m)
                g_start(j + NBUF - 1, (m + NBUF - 1) % NBUF)
                s_add(j, m)
            return 0
        lax.fori_loop(0, (NCHUNK - (NBUF - 1)) // NBUF, body, 0)
        for m in range(NBUF - 1):
            j = NCHUNK - (NBUF - 1) + m
            g_wait(jnp.int32(j), j % NBUF)
            s_add(jnp.int32(j), j % NBUF)

        plsc.subcore_barrier()
        pltpu.sync_copy(acc.at[pl.ds(base, RPT)],
                        out.at[cid, pl.ds(base, RPT)])

    return agg


def _make_degree():
    out_type = [jax.ShapeDtypeStruct((NC, NP, DW), jnp.float32)]
    scratch = [
        pltpu.VMEM((NCHUNK, CHUNK), jnp.int32),
        pltpu.VMEM((CHUNK, DW), jnp.float32),
        pltpu.VMEM_SHARED((NP, DW), jnp.float32),
    ]

    @functools.partial(
        pl.kernel, mesh=plsc.VectorSubcoreMesh(**_SC_MESH),
        out_type=out_type, scratch_types=scratch,
        compiler_params=pltpu.CompilerParams(use_tc_tiling_on_sc=False))
    def deg(dsts, zeros_h, ones_h, cnt_out, dst_v, ones_v, dacc):
        cid = lax.axis_index("c")
        sid = lax.axis_index("s")
        base = sid * RPT

        pltpu.sync_copy(zeros_h, dacc.at[pl.ds(base, RPT)])
        pltpu.sync_copy(ones_h, ones_v)
        plsc.subcore_barrier()

        pltpu.sync_copy(dsts.at[sid], dst_v)

        def body(j, _):
            pltpu.sync_copy(ones_v, dacc.at[dst_v.at[j]], add=True)
            return 0
        lax.fori_loop(0, NCHUNK, body, 0)

        plsc.subcore_barrier()
        pltpu.sync_copy(dacc.at[pl.ds(base, RPT)],
                        cnt_out.at[cid, pl.ds(base, RPT)])

    return deg


_R = 10000


def _make_dense():
    grid = (N // _R,)
    hp = jax.lax.Precision.DEFAULT

    def body(acc_ref, cnt_ref, h_ref, wlr_ref, bl_ref, wp_ref,
             flag_ref, out_ref, p_ref):
        cnt = cnt_ref[0, :, 0:1]
        rinv = 1.0 / jnp.maximum(cnt, 1.0)
        mean = jnp.concatenate([acc_ref[0], acc_ref[1]], axis=1) * rinv
        z = jnp.concatenate([mean, h_ref[...]], axis=1)
        o = jnp.dot(z, wlr_ref[...], precision=hp,
                    preferred_element_type=jnp.float32) + bl_ref[...]
        o = jnp.where(flag_ref[0] > 0.5, jnp.maximum(o, 0.0), o)
        out_ref[...] = o
        p = jnp.dot(o, wp_ref[...], precision=hp,
                    preferred_element_type=jnp.float32)
        p_ref[0] = p[:, :HD]
        p_ref[1] = p[:, HD:]

    in_specs = [
        pl.BlockSpec((NC, _R, HD), lambda i: (0, i, 0)),
        pl.BlockSpec((NC, _R, DW), lambda i: (0, i, 0)),
        pl.BlockSpec((_R, D), lambda i: (i, 0)),
        pl.BlockSpec((2 * D, D), lambda i: (0, 0)),
        pl.BlockSpec((1, D), lambda i: (0, 0)),
        pl.BlockSpec((D, D), lambda i: (0, 0)),
        pl.BlockSpec(memory_space=pltpu.SMEM),
    ]
    out_shape = [jax.ShapeDtypeStruct((N, D), jnp.float32),
                 jax.ShapeDtypeStruct((NC, N, HD), jnp.float32)]
    out_specs = [pl.BlockSpec((_R, D), lambda i: (i, 0)),
                 pl.BlockSpec((NC, _R, HD), lambda i: (0, i, 0))]

    return pl.pallas_call(body, grid=grid, in_specs=in_specs,
                          out_specs=out_specs, out_shape=out_shape)


_AGG = _make_agg()
_DEGREE = _make_degree()
_DENSE = _make_dense()


def _pad(w, rows, cols):
    return jnp.pad(w, ((0, rows - w.shape[0]), (0, cols - w.shape[1])))


def kernel(x, edge_index, Wl0, bl0, Wr0, Wl1, bl1, Wr1, Wl2, bl2, Wr2,
           Wl3, bl3, Wr3):
    pad = EPAD - E
    src = jnp.concatenate(
        [edge_index[0], jnp.zeros((pad,), jnp.int32)]).reshape(
            NS, NCHUNK, CHUNK)
    dst = jnp.concatenate(
        [edge_index[1], jnp.full((pad,), N, jnp.int32)]).reshape(
            NS, NCHUNK, CHUNK)

    eye = jnp.eye(D, dtype=jnp.float32)
    wl_s = jnp.stack([Wl0.T, eye, _pad(Wl2.T, D, D), Wl3.T])
    wr_s = jnp.stack([Wr0.T, _pad(Wr1.T, D, D), _pad(Wr2.T, D, D), Wr3.T])
    wlr_s = jnp.concatenate([wl_s, wr_s], axis=1)
    bl_s = jnp.stack([bl0.reshape(1, D), _pad(bl1.reshape(1, H), 1, D),
                      bl2.reshape(1, D), bl3.reshape(1, D)])
    p_s = jnp.stack([_pad(Wl1.T, D, D), eye, eye, eye])
    fl_s = jnp.array([[1.0], [0.0], [1.0], [0.0]], jnp.float32)

    (cnt,) = _DEGREE(dst, jnp.zeros((RPT, DW), jnp.float32),
                     jnp.ones((CHUNK, DW), jnp.float32))

    def layer(k, carry):
        h, t = carry
        (acc,) = _AGG(t, src, dst)
        wlr = lax.dynamic_index_in_dim(wlr_s, k, 0, keepdims=False)
        bl = lax.dynamic_index_in_dim(bl_s, k, 0, keepdims=False)
        p = lax.dynamic_index_in_dim(p_s, k, 0, keepdims=False)
        fl = lax.dynamic_index_in_dim(fl_s, k, 0, keepdims=False)
        o, t_next = _DENSE(acc, cnt, h, wlr, bl, p, fl)
        return (o, t_next)

    x_split = jnp.stack([x[:, :HD], x[:, HD:]])
    h, _ = lax.fori_loop(0, 4, layer, (x, x_split))
    return h


# speedup vs baseline: 1.0142x; 1.0142x over previous
import functools

import jax
import jax.numpy as jnp
from jax import lax
from jax.experimental import pallas as pl
from jax.experimental.pallas import tpu as pltpu
from jax.experimental.pallas import tpu_sc as plsc

N = 10000
E = 320000
D = 128
H = 64
HD = D // 2

NC = 2
NS = 16
L = 16

CHUNK = 128
NBUF = 3
NCHUNK = 158
WPE = NCHUNK * CHUNK
EPAD = WPE * NS
NP = 10240
RPT = NP // NS
DW = 8

_SC_MESH = dict(core_axis_name="c", subcore_axis_name="s")


def _make_agg():
    out_type = [jax.ShapeDtypeStruct((NC, NP, HD), jnp.float32)]
    scratch = (
        [pltpu.VMEM((NCHUNK, CHUNK), jnp.int32)] * 2
        + [pltpu.VMEM((CHUNK, HD), jnp.float32)] * NBUF
        + [pltpu.VMEM_SHARED((NP, HD), jnp.float32)]
        + [pltpu.SemaphoreType.DMA] * (2 * NBUF)
    )

    @functools.partial(
        pl.kernel, mesh=plsc.VectorSubcoreMesh(**_SC_MESH),
        out_type=out_type, scratch_types=scratch,
        compiler_params=pltpu.CompilerParams(use_tc_tiling_on_sc=False))
    def agg(table, srcs, dsts, out, src_v, dst_v, *rest):
        rows = rest[:NBUF]
        acc = rest[NBUF]
        gsem = rest[NBUF + 1:2 * NBUF + 1]
        ssem = rest[2 * NBUF + 1:]
        cid = lax.axis_index("c")
        sid = lax.axis_index("s")
        base = sid * RPT
        half = table.at[cid]

        idx_cp = (pltpu.async_copy(srcs.at[sid], src_v, gsem[0]),
                  pltpu.async_copy(dsts.at[sid], dst_v, gsem[1]))

        def zrow(r, _):
            for i in range(HD // L):
                rows[0][r, pl.ds(i * L, L)] = jnp.zeros((L,), jnp.float32)
            return 0
        lax.fori_loop(0, CHUNK, zrow, 0)
        for k in range(RPT // CHUNK):
            pltpu.sync_copy(rows[0], acc.at[pl.ds(base + k * CHUNK, CHUNK)])
        for cp in idx_cp:
            cp.wait()
        plsc.subcore_barrier()

        def g_start(j, m):
            pltpu.make_async_copy(half.at[src_v.at[j]], rows[m],
                                  gsem[m]).start()

        def g_wait(j, m):
            pltpu.make_async_copy(half.at[src_v.at[j]], rows[m],
                                  gsem[m]).wait()

        def s_add(j, m):
            pltpu.sync_copy(rows[m], acc.at[dst_v.at[j]], add=True)

        for m in range(NBUF - 1):
            g_start(jnp.int32(m), m)

        def body(jj, _):
            j0 = jj * NBUF
            for m in range(NBUF):
                j = j0 + m
                g_wait(j, m)
                g_start(j + NBUF - 1, (m + NBUF - 1) % NBUF)
                s_add(j, m)
            return 0
        lax.fori_loop(0, (NCHUNK - (NBUF - 1)) // NBUF, body, 0)
        for m in range(NBUF - 1):
            j = NCHUNK - (NBUF - 1) + m
            g_wait(jnp.int32(j), j % NBUF)
            s_add(jnp.int32(j), j % NBUF)

        plsc.subcore_barrier()
        pltpu.sync_copy(acc.at[pl.ds(base, RPT)],
                        out.at[cid, pl.ds(base, RPT)])

    return agg


def _make_degree():
    out_type = [jax.ShapeDtypeStruct((NC, NP, DW), jnp.float32)]
    scratch = [
        pltpu.VMEM((NCHUNK, CHUNK), jnp.int32),
        pltpu.VMEM((CHUNK, DW), jnp.float32),
        pltpu.VMEM_SHARED((NP, DW), jnp.float32),
    ]

    @functools.partial(
        pl.kernel, mesh=plsc.VectorSubcoreMesh(**_SC_MESH),
        out_type=out_type, scratch_types=scratch,
        compiler_params=pltpu.CompilerParams(use_tc_tiling_on_sc=False))
    def deg(dsts, zeros_h, ones_h, cnt_out, dst_v, ones_v, dacc):
        cid = lax.axis_index("c")
        sid = lax.axis_index("s")
        base = sid * RPT

        pltpu.sync_copy(zeros_h, dacc.at[pl.ds(base, RPT)])
        pltpu.sync_copy(ones_h, ones_v)
        plsc.subcore_barrier()

        pltpu.sync_copy(dsts.at[sid], dst_v)

        def body(j, _):
            pltpu.sync_copy(ones_v, dacc.at[dst_v.at[j]], add=True)
            return 0
        lax.fori_loop(0, NCHUNK, body, 0)

        plsc.subcore_barrier()
        pltpu.sync_copy(dacc.at[pl.ds(base, RPT)],
                        cnt_out.at[cid, pl.ds(base, RPT)])

    return deg


_R = 5000


def _make_dense():
    grid = (N // _R,)
    hp = jax.lax.Precision.DEFAULT

    def body(acc_ref, cnt_ref, h_ref, wlr_ref, bl_ref, wp_ref,
             flag_ref, out_ref, p_ref):
        cnt = cnt_ref[0, :, 0:1]
        rinv = 1.0 / jnp.maximum(cnt, 1.0)
        mean = jnp.concatenate([acc_ref[0], acc_ref[1]], axis=1) * rinv
        z = jnp.concatenate([mean, h_ref[...]], axis=1)
        o = jnp.dot(z, wlr_ref[...], precision=hp,
                    preferred_element_type=jnp.float32) + bl_ref[...]
        o = jnp.where(flag_ref[0] > 0.5, jnp.maximum(o, 0.0), o)
        out_ref[...] = o
        p = jnp.dot(o, wp_ref[...], precision=hp,
                    preferred_element_type=jnp.float32)
        p_ref[0] = p[:, :HD]
        p_ref[1] = p[:, HD:]

    in_specs = [
        pl.BlockSpec((NC, _R, HD), lambda i: (0, i, 0)),
        pl.BlockSpec((NC, _R, DW), lambda i: (0, i, 0)),
        pl.BlockSpec((_R, D), lambda i: (i, 0)),
        pl.BlockSpec((2 * D, D), lambda i: (0, 0)),
        pl.BlockSpec((1, D), lambda i: (0, 0)),
        pl.BlockSpec((D, D), lambda i: (0, 0)),
        pl.BlockSpec(memory_space=pltpu.SMEM),
    ]
    out_shape = [jax.ShapeDtypeStruct((N, D), jnp.float32),
                 jax.ShapeDtypeStruct((NC, N, HD), jnp.float32)]
    out_specs = [pl.BlockSpec((_R, D), lambda i: (i, 0)),
                 pl.BlockSpec((NC, _R, HD), lambda i: (0, i, 0))]

    return pl.pallas_call(body, grid=grid, in_specs=in_specs,
                          out_specs=out_specs, out_shape=out_shape)


_AGG = _make_agg()
_DEGREE = _make_degree()
_DENSE = _make_dense()


def _pad(w, rows, cols):
    return jnp.pad(w, ((0, rows - w.shape[0]), (0, cols - w.shape[1])))


def kernel(x, edge_index, Wl0, bl0, Wr0, Wl1, bl1, Wr1, Wl2, bl2, Wr2,
           Wl3, bl3, Wr3):
    pad = EPAD - E
    src = jnp.concatenate(
        [edge_index[0], jnp.zeros((pad,), jnp.int32)]).reshape(
            NS, NCHUNK, CHUNK)
    dst = jnp.concatenate(
        [edge_index[1], jnp.full((pad,), N, jnp.int32)]).reshape(
            NS, NCHUNK, CHUNK)

    eye = jnp.eye(D, dtype=jnp.float32)
    wl_s = jnp.stack([Wl0.T, eye, _pad(Wl2.T, D, D), Wl3.T])
    wr_s = jnp.stack([Wr0.T, _pad(Wr1.T, D, D), _pad(Wr2.T, D, D), Wr3.T])
    wlr_s = jnp.concatenate([wl_s, wr_s], axis=1)
    bl_s = jnp.stack([bl0.reshape(1, D), _pad(bl1.reshape(1, H), 1, D),
                      bl2.reshape(1, D), bl3.reshape(1, D)])
    p_s = jnp.stack([_pad(Wl1.T, D, D), eye, eye, eye])
    fl_s = jnp.array([[1.0], [0.0], [1.0], [0.0]], jnp.float32)

    (cnt,) = _DEGREE(dst, jnp.zeros((RPT, DW), jnp.float32),
                     jnp.ones((CHUNK, DW), jnp.float32))

    def layer(k, carry):
        h, t = carry
        (acc,) = _AGG(t, src, dst)
        wlr = lax.dynamic_index_in_dim(wlr_s, k, 0, keepdims=False)
        bl = lax.dynamic_index_in_dim(bl_s, k, 0, keepdims=False)
        p = lax.dynamic_index_in_dim(p_s, k, 0, keepdims=False)
        fl = lax.dynamic_index_in_dim(fl_s, k, 0, keepdims=False)
        o, t_next = _DENSE(acc, cnt, h, wlr, bl, p, fl)
        return (o, t_next)

    x_split = jnp.stack([x[:, :HD], x[:, HD:]])
    h, _ = lax.fori_loop(0, 4, layer, (x, x_split))
    return h
